# Initial kernel scaffold; baseline (speedup 1.0000x reference)
import jax
import jax.numpy as jnp
from jax.experimental import pallas as pl


def kernel(X, A, W1, b1, W2, b2, W3, b3):
    raise NotImplementedError("write your pallas kernel here")


# speedup vs baseline: 1.6917x; 1.6917x over previous
import jax
import jax.numpy as jnp
from jax.experimental import pallas as pl
from jax.experimental.pallas import tpu as pltpu

B, N, T, E = 8, 1024, 12, 8
IN_DIM, HID, OUT, K = T * E, 64, 32, 3


def _fused_kernel(a_ref, x_ref, w1_ref, b1_ref, w2_ref, b2_ref, w3_ref,
                  b3_ref, out_ref):
    a = a_ref[0]

    row = jax.lax.broadcasted_iota(jnp.int32, (N, N), 0)
    col = jax.lax.broadcasted_iota(jnp.int32, (N, N), 1)
    a_nd = jnp.where(row == col, 0.0, a)
    deg = jnp.sum(a_nd, axis=1, keepdims=True)
    dinv = jnp.where(deg > 0, jax.lax.rsqrt(jnp.maximum(deg, 1e-12)), 0.0)
    L = -(dinv * a_nd) * dinv.reshape(1, N)

    def matmul(p, q):
        return jax.lax.dot_general(
            p, q, (((1,), (0,)), ((), ())),
            preferred_element_type=jnp.float32)

    def cheb(h, w_ref, b_ref):
        t1 = matmul(L, h)
        t2 = 2.0 * matmul(L, t1) - h
        out = (matmul(h, w_ref[0]) + matmul(t1, w_ref[1])
               + matmul(t2, w_ref[2]) + b_ref[0])
        return out

    h = x_ref[0]
    h = jnp.maximum(cheb(h, w1_ref, b1_ref), 0.0)
    h = jnp.maximum(cheb(h, w2_ref, b2_ref), 0.0)
    out_ref[0] = cheb(h, w3_ref, b3_ref)


def kernel(X, A, W1, b1, W2, b2, W3, b3):
    x = X.reshape(B, N, IN_DIM)
    b1r = b1.reshape(1, HID)
    b2r = b2.reshape(1, HID)
    b3r = b3.reshape(1, OUT)

    full = lambda *s: pl.BlockSpec(s, lambda b: (0,) * len(s))
    return pl.pallas_call(
        _fused_kernel,
        grid=(B,),
        in_specs=[
            pl.BlockSpec((1, N, N), lambda b: (b, 0, 0)),
            pl.BlockSpec((1, N, IN_DIM), lambda b: (b, 0, 0)),
            full(K, IN_DIM, HID),
            full(1, HID),
            full(K, HID, HID),
            full(1, HID),
            full(K, HID, OUT),
            full(1, OUT),
        ],
        out_specs=pl.BlockSpec((1, N, OUT), lambda b: (b, 0, 0)),
        out_shape=jax.ShapeDtypeStruct((B, N, OUT), jnp.float32),
        compiler_params=pltpu.CompilerParams(
            dimension_semantics=("arbitrary",),
        ),
    )(A, x, W1, b1r, W2, b2r, W3, b3r)
